# baseline (speedup 1.0000x reference)
import functools

import jax
import jax.numpy as jnp
from jax import lax
from jax.experimental import pallas as pl
from jax.experimental.pallas import tpu as pltpu
from jax.experimental.pallas import tpu_sc as plsc

B = 16384
NF = 26
ND = 13
VOCAB = 1000000
NW = 32
RPW = B // NW
CPW = RPW // 16
GATH = RPW * NF


def _sc_call(table, ids, dense_f, w):
    mesh = plsc.VectorSubcoreMesh(core_axis_name="c", subcore_axis_name="s")

    @functools.partial(
        pl.kernel,
        mesh=mesh,
        out_type=jax.ShapeDtypeStruct((B,), jnp.float32),
        compiler_params=pltpu.CompilerParams(needs_layout_passes=False,
                                             skip_device_barrier=True),
        scratch_types=[
            pltpu.VMEM((GATH,), jnp.int32),
            pltpu.VMEM((GATH,), jnp.int32),
            pltpu.VMEM((GATH,), jnp.float32),
            pltpu.VMEM((RPW * ND,), jnp.float32),
            pltpu.VMEM((48,), jnp.float32),
            pltpu.VMEM((RPW,), jnp.float32),
            pltpu.SemaphoreType.DMA,
        ],
    )
    def k(table_h, ids_h, dense_h, w_h, out_h,
          ids_v, idx_v, vals_v, dense_v, w_v, out_v, sem):
        wid = lax.axis_index("s") * 2 + lax.axis_index("c")
        base = wid * RPW
        pltpu.sync_copy(ids_h.at[pl.ds(base * NF, GATH)], ids_v)
        pltpu.sync_copy(dense_h.at[pl.ds(base * ND, RPW * ND)], dense_v)
        pltpu.sync_copy(w_h, w_v)

        iota = lax.iota(jnp.int32, 16)

        def build(c, _):
            src = iota * NF + c * (16 * NF)
            for f in range(NF):
                g = plsc.load_gather(ids_v, [src + f])
                idx_v[pl.ds(f * RPW + c * 16, 16)] = g + f * VOCAB
            return 0

        lax.fori_loop(0, CPW, build, 0)

        pltpu.async_copy(table_h.at[idx_v], vals_v, sem).wait()

        wb = plsc.load_gather(w_v, [jnp.full((16,), 39, jnp.int32)])
        wsp = [plsc.load_gather(w_v, [jnp.full((16,), f, jnp.int32)])
               for f in range(NF)]
        wdn = [plsc.load_gather(w_v, [jnp.full((16,), NF + d, jnp.int32)])
               for d in range(ND)]

        def reduce(c, _):
            acc = wb
            for f in range(NF):
                v = vals_v[pl.ds(f * RPW + c * 16, 16)]
                acc = acc + v * wsp[f]
            dsrc = iota * ND + c * (16 * ND)
            for d in range(ND):
                dv = plsc.load_gather(dense_v, [dsrc + d])
                acc = acc + dv * wdn[d]
            out_v[pl.ds(c * 16, 16)] = acc
            return 0

        lax.fori_loop(0, CPW, reduce, 0)

        pltpu.sync_copy(out_v, out_h.at[pl.ds(base, RPW)])

    return k(table, ids, dense_f, w)


def kernel(sparse_ids, dense, emb_table, fc_w, fc_b):
    ids_flat = sparse_ids.reshape(-1)
    dense_flat = dense.reshape(-1)
    table_flat = emb_table.reshape(-1)
    w = jnp.concatenate([fc_w.reshape(-1), fc_b.reshape(-1),
                         jnp.zeros((8,), jnp.float32)])
    out = _sc_call(table_flat, ids_flat, dense_flat, w)
    return out.reshape(B, 1)
